# Initial kernel scaffold; baseline (speedup 1.0000x reference)
import functools

import jax
import jax.numpy as jnp
from jax import lax
from jax.experimental import pallas as pl
from jax.experimental.pallas import tpu as pltpu
from jax.experimental.pallas import tpu_sc as plsc

N = 10000
E = 320000
D = 128

NC = 2
NS = 16
NW = NC * NS

EPT = E // NW
C = 125
CHUNKS = EPT // C
RPT = N // NS

_mesh = plsc.VectorSubcoreMesh(core_axis_name="c", subcore_axis_name="s")


@functools.partial(
    pl.kernel,
    out_type=[
        jax.ShapeDtypeStruct((NC, N, 16), jnp.float32),
        jax.ShapeDtypeStruct((NC, N, 16), jnp.float32),
    ],
    mesh=_mesh,
    scratch_types=[
        pltpu.VMEM((CHUNKS, C), jnp.int32),
        pltpu.VMEM((CHUNKS, C), jnp.int32),
        pltpu.VMEM((C, 16), jnp.float32),
        pltpu.VMEM_SHARED((N, 16), jnp.float32),
        pltpu.VMEM_SHARED((N, 16), jnp.float32),
    ],
)
def _deg_kernel(src_hbm, dst_hbm, ones_hbm, zeros_hbm,
                odeg_hbm, ideg_hbm,
                src_v, dst_v, ones_v, odeg_s, ideg_s):
    cid = lax.axis_index("c")
    sid = lax.axis_index("s")
    wid = cid * NS + sid

    pltpu.sync_copy(src_hbm.at[wid], src_v)
    pltpu.sync_copy(dst_hbm.at[wid], dst_v)
    pltpu.sync_copy(ones_hbm, ones_v)
    pltpu.sync_copy(zeros_hbm.at[pl.ds(sid * RPT, RPT)],
                    odeg_s.at[pl.ds(sid * RPT, RPT)])
    pltpu.sync_copy(zeros_hbm.at[pl.ds(sid * RPT, RPT)],
                    ideg_s.at[pl.ds(sid * RPT, RPT)])
    plsc.subcore_barrier()

    def step(j, carry):
        pltpu.sync_copy(ones_v, odeg_s.at[src_v.at[j]], add=True)
        pltpu.sync_copy(ones_v, ideg_s.at[dst_v.at[j]], add=True)
        return carry

    lax.fori_loop(0, CHUNKS, step, 0)
    plsc.subcore_barrier()

    pltpu.sync_copy(odeg_s.at[pl.ds(sid * RPT, RPT)],
                    odeg_hbm.at[cid, pl.ds(sid * RPT, RPT)])
    pltpu.sync_copy(ideg_s.at[pl.ds(sid * RPT, RPT)],
                    ideg_hbm.at[cid, pl.ds(sid * RPT, RPT)])


@functools.partial(
    pl.kernel,
    out_type=jax.ShapeDtypeStruct((NC, N, D), jnp.float32),
    mesh=_mesh,
    scratch_types=[
        pltpu.VMEM((CHUNKS, C), jnp.int32),
        pltpu.VMEM((CHUNKS, C), jnp.int32),
        pltpu.VMEM((C, D), jnp.float32),
        pltpu.VMEM_SHARED((N, D), jnp.float32),
    ],
)
def _agg_kernel(hw_hbm, src_hbm, dst_hbm, zeros_hbm, out_hbm,
                src_v, dst_v, rows_v, acc_s):
    cid = lax.axis_index("c")
    sid = lax.axis_index("s")
    wid = cid * NS + sid

    pltpu.sync_copy(src_hbm.at[wid], src_v)
    pltpu.sync_copy(dst_hbm.at[wid], dst_v)
    pltpu.sync_copy(zeros_hbm.at[pl.ds(sid * RPT, RPT)],
                    acc_s.at[pl.ds(sid * RPT, RPT)])
    plsc.subcore_barrier()

    def step(j, carry):
        pltpu.sync_copy(hw_hbm.at[src_v.at[j]], rows_v)
        pltpu.sync_copy(rows_v, acc_s.at[dst_v.at[j]], add=True)
        return carry

    lax.fori_loop(0, CHUNKS, step, 0)
    plsc.subcore_barrier()

    pltpu.sync_copy(acc_s.at[pl.ds(sid * RPT, RPT)],
                    out_hbm.at[cid, pl.ds(sid * RPT, RPT)])


_TC_BLK = 1000


def _hw_body(h_ref, w_ref, od_ref, o_ref):
    deg = od_ref[0] + od_ref[1]
    norm = lax.rsqrt(jnp.clip(deg[:, 0:1], 1.0, None))
    o_ref[...] = jnp.dot(h_ref[...] * norm, w_ref[...],
                         preferred_element_type=jnp.float32)


def _final_body(p_ref, id_ref, b_ref, o_ref):
    agg = p_ref[0] + p_ref[1]
    deg = id_ref[0] + id_ref[1]
    norm = lax.rsqrt(jnp.clip(deg[:, 0:1], 1.0, None))
    x = agg * norm + b_ref[...]
    o_ref[...] = jnp.where(x >= 0, x, 0.01 * x)


_hw_call = pl.pallas_call(
    _hw_body,
    grid=(N // _TC_BLK,),
    in_specs=[
        pl.BlockSpec((_TC_BLK, D), lambda i: (i, 0)),
        pl.BlockSpec((D, D), lambda i: (0, 0)),
        pl.BlockSpec((NC, _TC_BLK, 16), lambda i: (0, i, 0)),
    ],
    out_specs=pl.BlockSpec((_TC_BLK, D), lambda i: (i, 0)),
    out_shape=jax.ShapeDtypeStruct((N, D), jnp.float32),
)

_final_call = pl.pallas_call(
    _final_body,
    grid=(N // _TC_BLK,),
    in_specs=[
        pl.BlockSpec((NC, _TC_BLK, D), lambda i: (0, i, 0)),
        pl.BlockSpec((NC, _TC_BLK, 16), lambda i: (0, i, 0)),
        pl.BlockSpec((1, D), lambda i: (0, 0)),
    ],
    out_specs=pl.BlockSpec((_TC_BLK, D), lambda i: (i, 0)),
    out_shape=jax.ShapeDtypeStruct((N, D), jnp.float32),
)


def kernel(h, edge_index, W, b):
    src3 = edge_index[0].reshape(NW, CHUNKS, C)
    dst3 = edge_index[1].reshape(NW, CHUNKS, C)
    ones16 = jnp.ones((C, 16), jnp.float32)
    zeros16 = jnp.zeros((N, 16), jnp.float32)
    zerosD = jnp.zeros((N, D), jnp.float32)

    odeg_p, ideg_p = _deg_kernel(src3, dst3, ones16, zeros16)
    hw = _hw_call(h, W, odeg_p)
    partials = _agg_kernel(hw, src3, dst3, zerosD)
    return _final_call(partials, ideg_p, b.reshape(1, D))


# speedup vs baseline: 5.8436x; 5.8436x over previous
import functools

import jax
import jax.numpy as jnp
from jax import lax
from jax.experimental import pallas as pl
from jax.experimental.pallas import tpu as pltpu
from jax.experimental.pallas import tpu_sc as plsc

N = 10000
E = 320000
D = 128

NC = 2
NS = 16
NW = NC * NS

EPT = E // NW
C = 125
CHUNKS = EPT // C
NP = 10240
RPT = NP // NS

def _deg_body(src_hbm, dst_hbm, ones_hbm, zeros_hbm,
                odeg_hbm, ideg_hbm,
                sidx_v, didx_v, ones_v, odeg_s, ideg_s):
    cid = lax.axis_index("c")
    sid = lax.axis_index("s")
    wid = cid * NS + sid

    pltpu.sync_copy(ones_hbm, ones_v)
    pltpu.sync_copy(zeros_hbm.at[pl.ds(sid * RPT, RPT)],
                    odeg_s.at[pl.ds(sid * RPT, RPT)])
    pltpu.sync_copy(zeros_hbm.at[pl.ds(sid * RPT, RPT)],
                    ideg_s.at[pl.ds(sid * RPT, RPT)])
    plsc.subcore_barrier()

    def step(j, carry):
        pltpu.sync_copy(src_hbm.at[wid, j], sidx_v)
        pltpu.sync_copy(dst_hbm.at[wid, j], didx_v)
        pltpu.sync_copy(ones_v, odeg_s.at[sidx_v], add=True)
        pltpu.sync_copy(ones_v, ideg_s.at[didx_v], add=True)
        return carry

    lax.fori_loop(0, CHUNKS, step, 0)
    plsc.subcore_barrier()

    pltpu.sync_copy(odeg_s.at[pl.ds(sid * RPT, RPT)],
                    odeg_hbm.at[cid, pl.ds(sid * RPT, RPT)])
    pltpu.sync_copy(ideg_s.at[pl.ds(sid * RPT, RPT)],
                    ideg_hbm.at[cid, pl.ds(sid * RPT, RPT)])


def _agg_body(hw_hbm, src_hbm, dst_hbm, zeros_hbm, out_hbm,
                sidx_v, didx_v, rows_v, acc_s):
    cid = lax.axis_index("c")
    sid = lax.axis_index("s")
    wid = cid * NS + sid

    pltpu.sync_copy(zeros_hbm.at[pl.ds(sid * RPT, RPT)],
                    acc_s.at[pl.ds(sid * RPT, RPT)])
    plsc.subcore_barrier()

    def step(j, carry):
        pltpu.sync_copy(src_hbm.at[wid, j], sidx_v)
        pltpu.sync_copy(dst_hbm.at[wid, j], didx_v)
        pltpu.sync_copy(hw_hbm.at[sidx_v], rows_v)
        pltpu.sync_copy(rows_v, acc_s.at[didx_v], add=True)
        return carry

    lax.fori_loop(0, CHUNKS, step, 0)
    plsc.subcore_barrier()

    pltpu.sync_copy(acc_s.at[pl.ds(sid * RPT, RPT)],
                    out_hbm.at[cid, pl.ds(sid * RPT, RPT)])


@functools.cache
def _sc_kernels():
    mesh = plsc.VectorSubcoreMesh(core_axis_name="c", subcore_axis_name="s",
                                  num_cores=NC, num_subcores=NS)
    deg = pl.kernel(
        _deg_body,
        out_type=[
            jax.ShapeDtypeStruct((NC, NP, 16), jnp.float32),
            jax.ShapeDtypeStruct((NC, NP, 16), jnp.float32),
        ],
        mesh=mesh,
        scratch_types=[
            pltpu.VMEM((C,), jnp.int32),
            pltpu.VMEM((C,), jnp.int32),
            pltpu.VMEM((C, 16), jnp.float32),
            pltpu.VMEM_SHARED((NP, 16), jnp.float32),
            pltpu.VMEM_SHARED((NP, 16), jnp.float32),
        ],
        compiler_params=pltpu.CompilerParams(use_tc_tiling_on_sc=False),
    )
    agg = pl.kernel(
        _agg_body,
        out_type=jax.ShapeDtypeStruct((NC, NP, D), jnp.float32),
        mesh=mesh,
        scratch_types=[
            pltpu.VMEM((C,), jnp.int32),
            pltpu.VMEM((C,), jnp.int32),
            pltpu.VMEM((C, D), jnp.float32),
            pltpu.VMEM_SHARED((NP, D), jnp.float32),
        ],
        compiler_params=pltpu.CompilerParams(use_tc_tiling_on_sc=False),
    )
    return deg, agg


_TC_BLK = 1000


def _hw_body(h_ref, w_ref, od_ref, o_ref):
    deg = od_ref[0] + od_ref[1]
    norm = lax.rsqrt(jnp.clip(deg[:, 0:1], 1.0, None))
    o_ref[...] = jnp.dot(h_ref[...] * norm, w_ref[...],
                         preferred_element_type=jnp.float32)


def _final_body(p_ref, id_ref, b_ref, o_ref):
    agg = p_ref[0] + p_ref[1]
    deg = id_ref[0] + id_ref[1]
    norm = lax.rsqrt(jnp.clip(deg[:, 0:1], 1.0, None))
    x = agg * norm + b_ref[...]
    o_ref[...] = jnp.where(x >= 0, x, 0.01 * x)


_hw_call = pl.pallas_call(
    _hw_body,
    grid=(N // _TC_BLK,),
    in_specs=[
        pl.BlockSpec((_TC_BLK, D), lambda i: (i, 0)),
        pl.BlockSpec((D, D), lambda i: (0, 0)),
        pl.BlockSpec((NC, _TC_BLK, 16), lambda i: (0, i, 0)),
    ],
    out_specs=pl.BlockSpec((_TC_BLK, D), lambda i: (i, 0)),
    out_shape=jax.ShapeDtypeStruct((N, D), jnp.float32),
)

_final_call = pl.pallas_call(
    _final_body,
    grid=(N // _TC_BLK,),
    in_specs=[
        pl.BlockSpec((NC, _TC_BLK, D), lambda i: (0, i, 0)),
        pl.BlockSpec((NC, _TC_BLK, 16), lambda i: (0, i, 0)),
        pl.BlockSpec((1, D), lambda i: (0, 0)),
    ],
    out_specs=pl.BlockSpec((_TC_BLK, D), lambda i: (i, 0)),
    out_shape=jax.ShapeDtypeStruct((N, D), jnp.float32),
)


def kernel(h, edge_index, W, b):
    src3 = edge_index[0].reshape(NW, CHUNKS, C)
    dst3 = edge_index[1].reshape(NW, CHUNKS, C)
    ones16 = jnp.ones((C, 16), jnp.float32)
    zeros16 = jnp.zeros((NP, 16), jnp.float32)
    zerosD = jnp.zeros((NP, D), jnp.float32)

    deg_kernel, agg_kernel = _sc_kernels()
    odeg_p, ideg_p = deg_kernel(src3, dst3, ones16, zeros16)
    hw = _hw_call(h, W, odeg_p)
    partials = agg_kernel(hw, src3, dst3, zerosD)
    return _final_call(partials, ideg_p, b.reshape(1, D))


# speedup vs baseline: 10.8966x; 1.8647x over previous
import functools

import jax
import jax.numpy as jnp
from jax import lax
from jax.experimental import pallas as pl
from jax.experimental.pallas import tpu as pltpu
from jax.experimental.pallas import tpu_sc as plsc

N = 10000
E = 320000
D = 128

NC = 2
NS = 16
NW = NC * NS

EPT = E // NW
C = 125
CHUNKS = EPT // C
NP = 10240
RPT = NP // NS

def _deg_body(idx_hbm, ones_hbm, zeros_hbm,
                odeg_hbm, ideg_hbm,
                idx_v, ones_v, odeg_s, ideg_s, psem, ssem):
    cid = lax.axis_index("c")
    sid = lax.axis_index("s")
    wid = cid * NS + sid

    cps = [
        pltpu.async_copy(idx_hbm.at[wid], idx_v, psem),
        pltpu.async_copy(ones_hbm, ones_v, psem),
        pltpu.async_copy(zeros_hbm.at[pl.ds(sid * RPT, RPT)],
                         odeg_s.at[pl.ds(sid * RPT, RPT)], psem),
        pltpu.async_copy(zeros_hbm.at[pl.ds(sid * RPT, RPT)],
                         ideg_s.at[pl.ds(sid * RPT, RPT)], psem),
    ]
    for cp in cps:
        cp.wait()
    plsc.subcore_barrier()

    @pl.loop(0, CHUNKS, step=4)
    def step(p):
        fired = []
        for q in range(4):
            fired.append(pltpu.async_copy(
                ones_v, odeg_s.at[idx_v.at[p + q, 0]], ssem, add=True))
            fired.append(pltpu.async_copy(
                ones_v, ideg_s.at[idx_v.at[p + q, 1]], ssem, add=True))
        for cp in fired:
            cp.wait()

    plsc.subcore_barrier()

    pltpu.sync_copy(odeg_s.at[pl.ds(sid * RPT, RPT)],
                    odeg_hbm.at[cid, pl.ds(sid * RPT, RPT)])
    pltpu.sync_copy(ideg_s.at[pl.ds(sid * RPT, RPT)],
                    ideg_hbm.at[cid, pl.ds(sid * RPT, RPT)])


def _agg_body(hw_hbm, idx_hbm, zeros_hbm, out_hbm,
                idx_a, idx_b, rows_a, rows_b, acc_s,
                psem, isem_a, isem_b, gsem_a, gsem_b):
    cid = lax.axis_index("c")
    sid = lax.axis_index("s")
    wid = cid * NS + sid

    zcp = pltpu.async_copy(zeros_hbm.at[pl.ds(sid * RPT, RPT)],
                           acc_s.at[pl.ds(sid * RPT, RPT)], psem)
    pltpu.sync_copy(idx_hbm.at[wid, 0], idx_a)
    pltpu.async_copy(hw_hbm.at[idx_a.at[0]], rows_a, gsem_a)
    pltpu.async_copy(idx_hbm.at[wid, 1], idx_b, isem_b)
    zcp.wait()
    plsc.subcore_barrier()

    @pl.loop(0, CHUNKS - 2, step=2)
    def pair(p):
        pltpu.make_async_copy(hw_hbm.at[idx_a.at[0]], rows_a, gsem_a).wait()
        pltpu.make_async_copy(idx_hbm.at[wid, p + 1], idx_b, isem_b).wait()
        pltpu.async_copy(hw_hbm.at[idx_b.at[0]], rows_b, gsem_b)
        pltpu.sync_copy(rows_a, acc_s.at[idx_a.at[1]], add=True)
        pltpu.async_copy(idx_hbm.at[wid, p + 2], idx_a, isem_a)
        pltpu.make_async_copy(hw_hbm.at[idx_b.at[0]], rows_b, gsem_b).wait()
        pltpu.make_async_copy(idx_hbm.at[wid, p + 2], idx_a, isem_a).wait()
        pltpu.async_copy(hw_hbm.at[idx_a.at[0]], rows_a, gsem_a)
        pltpu.sync_copy(rows_b, acc_s.at[idx_b.at[1]], add=True)
        pltpu.async_copy(idx_hbm.at[wid, p + 3], idx_b, isem_b)

    p = CHUNKS - 2
    pltpu.make_async_copy(hw_hbm.at[idx_a.at[0]], rows_a, gsem_a).wait()
    pltpu.make_async_copy(idx_hbm.at[wid, p + 1], idx_b, isem_b).wait()
    pltpu.async_copy(hw_hbm.at[idx_b.at[0]], rows_b, gsem_b)
    pltpu.sync_copy(rows_a, acc_s.at[idx_a.at[1]], add=True)
    pltpu.make_async_copy(hw_hbm.at[idx_b.at[0]], rows_b, gsem_b).wait()
    pltpu.sync_copy(rows_b, acc_s.at[idx_b.at[1]], add=True)

    plsc.subcore_barrier()

    pltpu.sync_copy(acc_s.at[pl.ds(sid * RPT, RPT)],
                    out_hbm.at[cid, pl.ds(sid * RPT, RPT)])


@functools.cache
def _sc_kernels():
    mesh = plsc.VectorSubcoreMesh(core_axis_name="c", subcore_axis_name="s",
                                  num_cores=NC, num_subcores=NS)
    deg = pl.kernel(
        _deg_body,
        out_type=[
            jax.ShapeDtypeStruct((NC, NP, 16), jnp.float32),
            jax.ShapeDtypeStruct((NC, NP, 16), jnp.float32),
        ],
        mesh=mesh,
        scratch_types=[
            pltpu.VMEM((CHUNKS, 2, C), jnp.int32),
            pltpu.VMEM((C, 16), jnp.float32),
            pltpu.VMEM_SHARED((NP, 16), jnp.float32),
            pltpu.VMEM_SHARED((NP, 16), jnp.float32),
            pltpu.SemaphoreType.DMA,
            pltpu.SemaphoreType.DMA,
        ],
        compiler_params=pltpu.CompilerParams(use_tc_tiling_on_sc=False),
    )
    agg = pl.kernel(
        _agg_body,
        out_type=jax.ShapeDtypeStruct((NC, NP, D), jnp.float32),
        mesh=mesh,
        scratch_types=[
            pltpu.VMEM((2, C), jnp.int32),
            pltpu.VMEM((2, C), jnp.int32),
            pltpu.VMEM((C, D), jnp.float32),
            pltpu.VMEM((C, D), jnp.float32),
            pltpu.VMEM_SHARED((NP, D), jnp.float32),
            pltpu.SemaphoreType.DMA,
            pltpu.SemaphoreType.DMA,
            pltpu.SemaphoreType.DMA,
            pltpu.SemaphoreType.DMA,
            pltpu.SemaphoreType.DMA,
        ],
        compiler_params=pltpu.CompilerParams(use_tc_tiling_on_sc=False),
    )
    return deg, agg


_TC_BLK = 1000


def _hw_body(h_ref, w_ref, od_ref, o_ref):
    deg = od_ref[0] + od_ref[1]
    norm = lax.rsqrt(jnp.clip(deg[:, 0:1], 1.0, None))
    o_ref[...] = jnp.dot(h_ref[...] * norm, w_ref[...],
                         preferred_element_type=jnp.float32)


def _final_body(p_ref, id_ref, b_ref, o_ref):
    agg = p_ref[0] + p_ref[1]
    deg = id_ref[0] + id_ref[1]
    norm = lax.rsqrt(jnp.clip(deg[:, 0:1], 1.0, None))
    x = agg * norm + b_ref[...]
    o_ref[...] = jnp.where(x >= 0, x, 0.01 * x)


_hw_call = pl.pallas_call(
    _hw_body,
    grid=(N // _TC_BLK,),
    in_specs=[
        pl.BlockSpec((_TC_BLK, D), lambda i: (i, 0)),
        pl.BlockSpec((D, D), lambda i: (0, 0)),
        pl.BlockSpec((NC, _TC_BLK, 16), lambda i: (0, i, 0)),
    ],
    out_specs=pl.BlockSpec((_TC_BLK, D), lambda i: (i, 0)),
    out_shape=jax.ShapeDtypeStruct((N, D), jnp.float32),
)

_final_call = pl.pallas_call(
    _final_body,
    grid=(N // _TC_BLK,),
    in_specs=[
        pl.BlockSpec((NC, _TC_BLK, D), lambda i: (0, i, 0)),
        pl.BlockSpec((NC, _TC_BLK, 16), lambda i: (0, i, 0)),
        pl.BlockSpec((1, D), lambda i: (0, 0)),
    ],
    out_specs=pl.BlockSpec((_TC_BLK, D), lambda i: (i, 0)),
    out_shape=jax.ShapeDtypeStruct((N, D), jnp.float32),
)


def kernel(h, edge_index, W, b):
    idx4 = jnp.stack([edge_index[0].reshape(NW, CHUNKS, C),
                      edge_index[1].reshape(NW, CHUNKS, C)], axis=2)
    ones16 = jnp.ones((C, 16), jnp.float32)
    zeros16 = jnp.zeros((NP, 16), jnp.float32)
    zerosD = jnp.zeros((NP, D), jnp.float32)

    deg_kernel, agg_kernel = _sc_kernels()
    odeg_p, ideg_p = deg_kernel(idx4, ones16, zeros16)
    hw = _hw_call(h, W, odeg_p)
    partials = agg_kernel(hw, idx4, zerosD)
    return _final_call(partials, ideg_p, b.reshape(1, D))
